# baseline (speedup 1.0000x reference)
import functools

import jax
import jax.numpy as jnp
from jax import lax
from jax.experimental import pallas as pl
from jax.experimental.pallas import tpu as pltpu
from jax.experimental.pallas import tpu_sc as plsc

V = 256
F_IN = 128
H = 64
W = 2 * H
E = 4096
NUM_LAYERS = 2
G3 = 3 * H

NC = 2
NS = 16
NW = NC * NS
EPW = E // NW

BV = 16
NBV = V // BV

CB = 8
NCB = V // CB


def _embed_body(adj_ref, ew_ref, xm_ref, csnz_ref):
    xb = jnp.dot(adj_ref[...], ew_ref[...], preferred_element_type=jnp.float32)
    xm_ref[...] = jnp.concatenate([xb, xb], axis=-1)
    xb3 = xb.reshape(BV, V, H)
    csb = jnp.sum(xb3, axis=1)
    nzb = jnp.sum((xb3 != 0.0).astype(jnp.float32), axis=1)
    csnz_ref[...] = jnp.concatenate([csb, nzb], axis=-1)


_embed_call = pl.pallas_call(
    _embed_body,
    grid=(NBV,),
    in_specs=[
        pl.BlockSpec((BV * V, F_IN), lambda i: (i, 0)),
        pl.BlockSpec((F_IN, H), lambda i: (0, 0)),
    ],
    out_specs=[
        pl.BlockSpec((BV * V, W), lambda i: (i, 0)),
        pl.BlockSpec((BV, W), lambda i: (i, 0)),
    ],
    out_shape=[
        jax.ShapeDtypeStruct((V * V, W), jnp.float32),
        jax.ShapeDtypeStruct((V, W), jnp.float32),
    ],
)


def _gather_body(xm_hbm, csnz_hbm, edges_hbm,
                 g1_hbm, g2_hbm, g3_hbm, g4_hbm,
                 eu_v, ev_v, fa_v, fb_v,
                 b1, b2, b3, b4, sem):
    wid = lax.axis_index("s") * NC + lax.axis_index("c")
    base = wid * EPW
    pltpu.sync_copy(edges_hbm.at[0, pl.ds(base, EPW)], eu_v)
    pltpu.sync_copy(edges_hbm.at[1, pl.ds(base, EPW)], ev_v)
    for c in range(EPW // 16):
        sl = pl.ds(c * 16, 16)
        u = eu_v[sl]
        v = ev_v[sl]
        fa_v[sl] = v * V + u
        fb_v[sl] = u * V + v
    cps = [
        pltpu.async_copy(xm_hbm.at[fa_v], b1, sem),
        pltpu.async_copy(xm_hbm.at[fb_v], b2, sem),
        pltpu.async_copy(csnz_hbm.at[eu_v], b3, sem),
        pltpu.async_copy(csnz_hbm.at[ev_v], b4, sem),
    ]
    for cp in cps:
        cp.wait()
    out_sl = pl.ds(base, EPW)
    pltpu.sync_copy(b1, g1_hbm.at[out_sl])
    pltpu.sync_copy(b2, g2_hbm.at[out_sl])
    pltpu.sync_copy(b3, g3_hbm.at[out_sl])
    pltpu.sync_copy(b4, g4_hbm.at[out_sl])


@functools.lru_cache(maxsize=None)
def _get_sc_gather():
  return pl.kernel(
    _gather_body,
    out_type=[jax.ShapeDtypeStruct((E, W), jnp.float32)] * 4,
    mesh=plsc.VectorSubcoreMesh(
        core_axis_name="c", subcore_axis_name="s", num_cores=NC, num_subcores=NS
    ),
    scratch_types=[
        pltpu.VMEM((EPW,), jnp.int32),
        pltpu.VMEM((EPW,), jnp.int32),
        pltpu.VMEM((EPW,), jnp.int32),
        pltpu.VMEM((EPW,), jnp.int32),
        pltpu.VMEM((EPW, W), jnp.float32),
        pltpu.VMEM((EPW, W), jnp.float32),
        pltpu.VMEM((EPW, W), jnp.float32),
        pltpu.VMEM((EPW, W), jnp.float32),
        pltpu.SemaphoreType.DMA,
    ],
  )


def _msg_body(g1_ref, g2_ref, g3_ref, g4_ref, w1_ref, w2_ref, w3_ref, out_ref):
    eemb = g1_ref[:, H:]
    xvu = g2_ref[:, H:]
    csu = g3_ref[:, :H]
    nzu = g3_ref[:, H:]
    csv = g4_ref[:, :H]
    nzv = g4_ref[:, H:]
    cnt_u = jnp.sum(nzu, axis=1, keepdims=True)
    cnt_v = jnp.sum(nzv, axis=1, keepdims=True)
    nz_xvu = jnp.sum((xvu != 0.0).astype(jnp.float32), axis=1, keepdims=True)
    nz_eemb = jnp.sum((eemb != 0.0).astype(jnp.float32), axis=1, keepdims=True)
    n_in = (cnt_u - nz_xvu) * (1.0 / H)
    n_in = jnp.where(n_in == 0.0, 1.0, n_in)
    n_out = (cnt_v - nz_eemb) * (1.0 / H)
    n_out = jnp.where(n_out == 0.0, 1.0, n_out)
    e_in = (csu - xvu) / n_in
    e_out = (csv - eemb) / n_out
    acc = jnp.dot(eemb, w1_ref[...], preferred_element_type=jnp.float32)
    acc += jnp.dot(e_in, w2_ref[...], preferred_element_type=jnp.float32)
    acc += jnp.dot(e_out, w3_ref[...], preferred_element_type=jnp.float32)
    msgs = jnp.maximum(acc, 0.0)
    out_ref[...] = jnp.concatenate([msgs, eemb], axis=-1)


_EB = 1024

_msg_mlp = pl.pallas_call(
    _msg_body,
    grid=(E // _EB,),
    in_specs=[pl.BlockSpec((_EB, W), lambda i: (i, 0))] * 4
    + [pl.BlockSpec((H, H), lambda i: (0, 0))] * 3,
    out_specs=pl.BlockSpec((_EB, W), lambda i: (i, 0)),
    out_shape=jax.ShapeDtypeStruct((E, W), jnp.float32),
)


def _scatter_body(msgs_hbm, edges_hbm, xm_hbm, eu_v, ev_v, fa_v, rows_v, sem):
    wid = lax.axis_index("s") * NC + lax.axis_index("c")
    base = wid * EPW
    pltpu.sync_copy(edges_hbm.at[0, pl.ds(base, EPW)], eu_v)
    pltpu.sync_copy(edges_hbm.at[1, pl.ds(base, EPW)], ev_v)
    for c in range(EPW // 16):
        sl = pl.ds(c * 16, 16)
        fa_v[sl] = ev_v[sl] * V + eu_v[sl]
    pltpu.sync_copy(msgs_hbm.at[pl.ds(base, EPW)], rows_v)
    pltpu.async_copy(rows_v, xm_hbm.at[fa_v], sem).wait()


@functools.lru_cache(maxsize=None)
def _get_sc_scatter():
  return pl.kernel(
    _scatter_body,
    out_type=(),
    mesh=plsc.VectorSubcoreMesh(
        core_axis_name="c", subcore_axis_name="s", num_cores=NC, num_subcores=NS
    ),
    scratch_types=[
        pltpu.VMEM((EPW,), jnp.int32),
        pltpu.VMEM((EPW,), jnp.int32),
        pltpu.VMEM((EPW,), jnp.int32),
        pltpu.VMEM((EPW, W), jnp.float32),
        pltpu.SemaphoreType.DMA,
    ],
  )


def _gru_body(with_stats, xm_ref, wih_ref, wh_ref, bih_ref, bhh_ref,
              out_ref, *rest):
    if with_stats:
        csnz_ref, h_ref = rest
    else:
        (h_ref,) = rest
    j = pl.program_id(0)

    @pl.when(j == 0)
    def _():
        h_ref[...] = jnp.zeros((V, H), jnp.float32)

    wih = wih_ref[...]
    bih = bih_ref[...]
    wh = wh_ref[...]
    bhh = bhh_ref[...]
    h = h_ref[...]
    for jj in range(CB):
        gi = jnp.dot(xm_ref[jj], wih,
                     preferred_element_type=jnp.float32) + bih
        gh = jnp.dot(h, wh, preferred_element_type=jnp.float32) + bhh
        rz = jax.nn.sigmoid(gi[:, :2 * H] + gh[:, :2 * H])
        r = rz[:, :H]
        z = rz[:, H:]
        n = jnp.tanh(gi[:, 2 * H:] + r * gh[:, 2 * H:])
        h = (1.0 - z) * n + z * h
        if with_stats:
            out_ref[jj] = jnp.concatenate([h, h], axis=-1)
            cs = jnp.sum(h, axis=0, keepdims=True)
            nzc = jnp.sum((h != 0.0).astype(jnp.float32), axis=0,
                          keepdims=True)
            csnz_ref[jj:jj + 1, :] = jnp.concatenate([cs, nzc], axis=-1)
        else:
            out_ref[jj] = h
    h_ref[...] = h


def _make_gru(with_stats):
    ow = W if with_stats else H
    out_specs = [pl.BlockSpec((CB, V, ow), lambda j: (j, 0, 0))]
    out_shape = [jax.ShapeDtypeStruct((V, V, ow), jnp.float32)]
    if with_stats:
        out_specs += [pl.BlockSpec((CB, W), lambda j: (j, 0))]
        out_shape += [jax.ShapeDtypeStruct((V, W), jnp.float32)]
    return pl.pallas_call(
        functools.partial(_gru_body, with_stats),
        grid=(NCB,),
        in_specs=[
            pl.BlockSpec((CB, V, W), lambda j: (j, 0, 0)),
            pl.BlockSpec((W, G3), lambda j: (0, 0)),
            pl.BlockSpec((H, G3), lambda j: (0, 0)),
            pl.BlockSpec((1, G3), lambda j: (0, 0)),
            pl.BlockSpec((1, G3), lambda j: (0, 0)),
        ],
        out_specs=out_specs,
        out_shape=out_shape,
        scratch_shapes=[pltpu.VMEM((V, H), jnp.float32)],
    )


_gru_stats = _make_gru(True)
_gru_final = _make_gru(False)


def kernel(adj_matrix, edges, emb_w, msg_w1, msg_w2, msg_w3,
           gru_wih, gru_whh, gru_bih, gru_bhh):
    adjT = jnp.swapaxes(adj_matrix, 0, 1).reshape(V * V, F_IN)
    emb_wT = emb_w.T
    wihT = gru_wih.T
    whT = gru_whh.T
    bih = gru_bih.reshape(1, G3)
    bhh = gru_bhh.reshape(1, G3)
    edges_t = edges.T.astype(jnp.int32)

    xm, csnz = _embed_call(adjT, emb_wT)
    out = None
    for layer in range(NUM_LAYERS):
        w1T = msg_w1[layer].T
        w2T = msg_w2[layer].T
        w3T = msg_w3[layer].T
        mref = jax.new_ref(xm)
        g1, g2, g3, g4 = _get_sc_gather()(mref, csnz, edges_t)
        msgs = _msg_mlp(g1, g2, g3, g4, w1T, w2T, w3T)
        _get_sc_scatter()(msgs, edges_t, mref)
        xm3 = jax.freeze(mref).reshape(V, V, W)
        if layer == 0:
            xmn, csnz = _gru_stats(xm3, wihT, whT, bih, bhh)
            xm = xmn.reshape(V * V, W)
        else:
            out = _gru_final(xm3, wihT, whT, bih, bhh)[0]
    return jnp.swapaxes(out, 0, 1)


# speedup vs baseline: 1.0017x; 1.0017x over previous
import functools

import jax
import jax.numpy as jnp
from jax import lax
from jax.experimental import pallas as pl
from jax.experimental.pallas import tpu as pltpu
from jax.experimental.pallas import tpu_sc as plsc

V = 256
F_IN = 128
H = 64
W = 2 * H
E = 4096
NUM_LAYERS = 2
G3 = 3 * H

NC = 2
NS = 16
NW = NC * NS
EPW = E // NW

BV = 16
NBV = V // BV

CB = 8
NCB = V // CB


def _embed_body(adj_ref, ew_ref, xm_ref, csnz_ref):
    xb = jnp.dot(adj_ref[...], ew_ref[...], preferred_element_type=jnp.float32)
    xm_ref[...] = jnp.concatenate([xb, xb], axis=-1)
    xb3 = xb.reshape(BV, V, H)
    csb = jnp.sum(xb3, axis=1)
    nzb = jnp.sum((xb3 != 0.0).astype(jnp.float32), axis=1)
    csnz_ref[...] = jnp.concatenate([csb, nzb], axis=-1)


_embed_call = pl.pallas_call(
    _embed_body,
    grid=(NBV,),
    in_specs=[
        pl.BlockSpec((BV * V, F_IN), lambda i: (i, 0)),
        pl.BlockSpec((F_IN, H), lambda i: (0, 0)),
    ],
    out_specs=[
        pl.BlockSpec((BV * V, W), lambda i: (i, 0)),
        pl.BlockSpec((BV, W), lambda i: (i, 0)),
    ],
    out_shape=[
        jax.ShapeDtypeStruct((V * V, W), jnp.float32),
        jax.ShapeDtypeStruct((V, W), jnp.float32),
    ],
)


def _gather_body(xm_hbm, csnz_hbm, edges_hbm,
                 g1_hbm, g2_hbm, g3_hbm, g4_hbm,
                 eu_v, ev_v, fa_v, fb_v,
                 b1, b2, b3, b4, sem):
    wid = lax.axis_index("s") * NC + lax.axis_index("c")
    base = wid * EPW
    pltpu.sync_copy(edges_hbm.at[0, pl.ds(base, EPW)], eu_v)
    pltpu.sync_copy(edges_hbm.at[1, pl.ds(base, EPW)], ev_v)
    for c in range(EPW // 16):
        sl = pl.ds(c * 16, 16)
        u = eu_v[sl]
        v = ev_v[sl]
        fa_v[sl] = v * V + u
        fb_v[sl] = u * V + v
    cps = [
        pltpu.async_copy(xm_hbm.at[fa_v], b1, sem),
        pltpu.async_copy(xm_hbm.at[fb_v], b2, sem),
        pltpu.async_copy(csnz_hbm.at[eu_v], b3, sem),
        pltpu.async_copy(csnz_hbm.at[ev_v], b4, sem),
    ]
    for cp in cps:
        cp.wait()
    out_sl = pl.ds(base, EPW)
    pltpu.sync_copy(b1, g1_hbm.at[out_sl])
    pltpu.sync_copy(b2, g2_hbm.at[out_sl])
    pltpu.sync_copy(b3, g3_hbm.at[out_sl])
    pltpu.sync_copy(b4, g4_hbm.at[out_sl])


@functools.lru_cache(maxsize=None)
def _get_sc_gather():
  return pl.kernel(
    _gather_body,
    out_type=[jax.ShapeDtypeStruct((E, W), jnp.float32)] * 4,
    mesh=plsc.VectorSubcoreMesh(
        core_axis_name="c", subcore_axis_name="s", num_cores=NC, num_subcores=NS
    ),
    scratch_types=[
        pltpu.VMEM((EPW,), jnp.int32),
        pltpu.VMEM((EPW,), jnp.int32),
        pltpu.VMEM((EPW,), jnp.int32),
        pltpu.VMEM((EPW,), jnp.int32),
        pltpu.VMEM((EPW, W), jnp.float32),
        pltpu.VMEM((EPW, W), jnp.float32),
        pltpu.VMEM((EPW, W), jnp.float32),
        pltpu.VMEM((EPW, W), jnp.float32),
        pltpu.SemaphoreType.DMA,
    ],
  )


def _msg_body(g1_ref, g2_ref, g3_ref, g4_ref, w1_ref, w2_ref, w3_ref, out_ref):
    eemb = g1_ref[:, H:]
    xvu = g2_ref[:, H:]
    csu = g3_ref[:, :H]
    nzu = g3_ref[:, H:]
    csv = g4_ref[:, :H]
    nzv = g4_ref[:, H:]
    cnt_u = jnp.sum(nzu, axis=1, keepdims=True)
    cnt_v = jnp.sum(nzv, axis=1, keepdims=True)
    nz_xvu = jnp.sum((xvu != 0.0).astype(jnp.float32), axis=1, keepdims=True)
    nz_eemb = jnp.sum((eemb != 0.0).astype(jnp.float32), axis=1, keepdims=True)
    n_in = (cnt_u - nz_xvu) * (1.0 / H)
    n_in = jnp.where(n_in == 0.0, 1.0, n_in)
    n_out = (cnt_v - nz_eemb) * (1.0 / H)
    n_out = jnp.where(n_out == 0.0, 1.0, n_out)
    e_in = (csu - xvu) / n_in
    e_out = (csv - eemb) / n_out
    acc = jnp.dot(eemb, w1_ref[...], preferred_element_type=jnp.float32)
    acc += jnp.dot(e_in, w2_ref[...], preferred_element_type=jnp.float32)
    acc += jnp.dot(e_out, w3_ref[...], preferred_element_type=jnp.float32)
    msgs = jnp.maximum(acc, 0.0)
    out_ref[...] = jnp.concatenate([msgs, eemb], axis=-1)


_EB = 1024

_msg_mlp = pl.pallas_call(
    _msg_body,
    grid=(E // _EB,),
    in_specs=[pl.BlockSpec((_EB, W), lambda i: (i, 0))] * 4
    + [pl.BlockSpec((H, H), lambda i: (0, 0))] * 3,
    out_specs=pl.BlockSpec((_EB, W), lambda i: (i, 0)),
    out_shape=jax.ShapeDtypeStruct((E, W), jnp.float32),
)


def _scatter_body(msgs_hbm, edges_hbm, xm_hbm, eu_v, ev_v, fa_v, rows_v, sem):
    wid = lax.axis_index("s") * NC + lax.axis_index("c")
    base = wid * EPW
    pltpu.sync_copy(edges_hbm.at[0, pl.ds(base, EPW)], eu_v)
    pltpu.sync_copy(edges_hbm.at[1, pl.ds(base, EPW)], ev_v)
    for c in range(EPW // 16):
        sl = pl.ds(c * 16, 16)
        fa_v[sl] = ev_v[sl] * V + eu_v[sl]
    pltpu.sync_copy(msgs_hbm.at[pl.ds(base, EPW)], rows_v)
    pltpu.async_copy(rows_v, xm_hbm.at[fa_v], sem).wait()


@functools.lru_cache(maxsize=None)
def _get_sc_scatter():
  return pl.kernel(
    _scatter_body,
    out_type=(),
    mesh=plsc.VectorSubcoreMesh(
        core_axis_name="c", subcore_axis_name="s", num_cores=NC, num_subcores=NS
    ),
    scratch_types=[
        pltpu.VMEM((EPW,), jnp.int32),
        pltpu.VMEM((EPW,), jnp.int32),
        pltpu.VMEM((EPW,), jnp.int32),
        pltpu.VMEM((EPW, W), jnp.float32),
        pltpu.SemaphoreType.DMA,
    ],
  )


def _gru_body(with_stats, xm_ref, wih_ref, wh_ref, bih_ref, bhh_ref,
              out_ref, *rest):
    if with_stats:
        csnz_ref, h_ref = rest
    else:
        (h_ref,) = rest
    j = pl.program_id(0)

    @pl.when(j == 0)
    def _():
        h_ref[...] = jnp.zeros((V, H), jnp.float32)

    wih = wih_ref[...]
    bih = bih_ref[...]
    wh = wh_ref[...]
    bhh = bhh_ref[...]
    h = h_ref[...]
    for jj in range(CB):
        gi = jnp.dot(xm_ref[jj], wih,
                     preferred_element_type=jnp.float32) + bih
        gh = jnp.dot(h, wh, preferred_element_type=jnp.float32) + bhh
        rz = jax.nn.sigmoid(gi[:, :2 * H] + gh[:, :2 * H])
        r = rz[:, :H]
        z = rz[:, H:]
        n = jnp.tanh(gi[:, 2 * H:] + r * gh[:, 2 * H:])
        h = (1.0 - z) * n + z * h
        if with_stats:
            out_ref[jj] = jnp.concatenate([h, h], axis=-1)
            cs = jnp.sum(h, axis=0, keepdims=True)
            nzc = jnp.sum((h != 0.0).astype(jnp.float32), axis=0,
                          keepdims=True)
            csnz_ref[jj:jj + 1, :] = jnp.concatenate([cs, nzc], axis=-1)
        else:
            out_ref[jj] = h
    h_ref[...] = h


def _make_gru(with_stats):
    ow = W if with_stats else H
    out_specs = [pl.BlockSpec((CB, V, ow), lambda j: (j, 0, 0))]
    out_shape = [jax.ShapeDtypeStruct((V, V, ow), jnp.float32)]
    if with_stats:
        out_specs += [pl.BlockSpec((CB, W), lambda j: (j, 0))]
        out_shape += [jax.ShapeDtypeStruct((V, W), jnp.float32)]
    return pl.pallas_call(
        functools.partial(_gru_body, with_stats),
        grid=(NCB,),
        in_specs=[
            pl.BlockSpec((CB, V, W), lambda j: (j, 0, 0)),
            pl.BlockSpec((W, G3), lambda j: (0, 0)),
            pl.BlockSpec((H, G3), lambda j: (0, 0)),
            pl.BlockSpec((1, G3), lambda j: (0, 0)),
            pl.BlockSpec((1, G3), lambda j: (0, 0)),
        ],
        out_specs=out_specs,
        out_shape=out_shape,
        scratch_shapes=[pltpu.VMEM((V, H), jnp.float32)],
    )


_gru_stats = _make_gru(True)
_gru_final = _make_gru(False)


def kernel(adj_matrix, edges, emb_w, msg_w1, msg_w2, msg_w3,
           gru_wih, gru_whh, gru_bih, gru_bhh):
    adjT = jnp.swapaxes(adj_matrix, 0, 1).reshape(V * V, F_IN)
    emb_wT = emb_w.T
    wihT = gru_wih.T
    whT = gru_whh.T
    bih = gru_bih.reshape(1, G3)
    bhh = gru_bhh.reshape(1, G3)
    edges_t = edges.T.astype(jnp.int32)

    xm, csnz = _embed_call(adjT, emb_wT)
    out = None
    for layer in range(NUM_LAYERS):
        w1T = msg_w1[layer].T
        w2T = msg_w2[layer].T
        w3T = msg_w3[layer].T
        g1, g2, g3, g4 = _get_sc_gather()(xm, csnz, edges_t)
        msgs = _msg_mlp(g1, g2, g3, g4, w1T, w2T, w3T)
        mref = jax.new_ref(xm)
        _get_sc_scatter()(msgs, edges_t, mref)
        xm3 = jax.freeze(mref).reshape(V, V, W)
        if layer == 0:
            xmn, csnz = _gru_stats(xm3, wihT, whT, bih, bhh)
            xm = xmn.reshape(V * V, W)
        else:
            out = _gru_final(xm3, wihT, whT, bih, bhh)[0]
    return jnp.swapaxes(out, 0, 1)
